# Initial kernel scaffold; baseline (speedup 1.0000x reference)
import jax
import jax.numpy as jnp
from jax.experimental import pallas as pl


def kernel(alphas, noise_u, idx):
    raise NotImplementedError("write your pallas kernel here")


# speedup vs baseline: 3.2908x; 3.2908x over previous
import functools

import jax
import jax.numpy as jnp
from jax import lax
from jax.experimental import pallas as pl
from jax.experimental.pallas import tpu as pltpu
from jax.experimental.pallas import tpu_sc as plsc

NN = 512
OPS = 32
E = NN * NN // 2
NSLOT = NN * NN
PAD = 2048
NC, NS, L = 2, 16, 16
NW = NC * NS
S = NSLOT // NW
CH = 16384
GB = 128
BLK = 2048


def _tc_body(a_ref, u_ref, w_ref, l_ref):
    i = pl.program_id(0)
    a = a_ref[...]
    u = jnp.clip(u_ref[...], 1e-6, 1.0 - 1e-6)
    w = u / (u + (1.0 - u) * jnp.exp(-a))
    is_pad = i >= E // BLK
    w_ref[...] = jnp.where(is_pad, 0.0, w)
    l_ref[...] = jnp.where(is_pad, 0.0, a)


def _sample_and_pad(alphas, noise_u):
    nblk = (E + PAD) // BLK
    last = E // BLK - 1
    return pl.pallas_call(
        _tc_body,
        grid=(nblk,),
        in_specs=[pl.BlockSpec((BLK, OPS), lambda i: (jnp.minimum(i, last), 0))] * 2,
        out_specs=[pl.BlockSpec((BLK, OPS), lambda i: (i, 0))] * 2,
        out_shape=[jax.ShapeDtypeStruct((E + PAD, OPS), jnp.float32)] * 2,
    )(alphas, noise_u)


def _sc_body(i0_hbm, i1_hbm, opw_hbm, alph_hbm, out_hbm, win, i0b, i1b, rows, sem):
    wid = lax.axis_index("s") * NC + lax.axis_index("c")
    base = wid * S
    iota = lax.broadcasted_iota(jnp.int32, (L,), 0)

    def init_row(j, carry):
        for k in range(GB // L):
            sent = E + ((j * GB + k * L + iota) & (PAD - 1))
            win[j, pl.ds(k * L, L)] = sent
        return carry

    lax.fori_loop(0, S // GB, init_row, 0)

    UNROLL = 8
    for c in range(E // CH):
        pltpu.sync_copy(i0_hbm.at[pl.ds(c * CH, CH)], i0b)
        pltpu.sync_copy(i1_hbm.at[pl.ds(c * CH, CH)], i1b)

        def scan_body(i, carry, c=c):
            for k in range(UNROLL):
                off = i * (UNROLL * L) + k * L
                v0 = i0b[pl.ds(off, L)]
                v1 = i1b[pl.ds(off, L)]
                rel = v0 * NN + v1 - base
                m = (rel >= 0) & (rel < S)
                relc = jnp.where(m, rel, 0)
                evec = (c * CH) + off + iota
                plsc.store_scatter(
                    win, [relc >> 7, relc & (GB - 1)], evec, mask=m)
            return carry

        lax.fori_loop(0, CH // (UNROLL * L), scan_body, 0)

    def emit(plane, src_hbm):
        def g_body(j, carry):
            pltpu.async_copy(src_hbm.at[win.at[j]], rows, sem).wait()
            pltpu.sync_copy(rows, out_hbm.at[plane, pl.ds(base + j * GB, GB)])
            return carry
        lax.fori_loop(0, S // GB, g_body, 0)

    emit(0, opw_hbm)
    emit(1, alph_hbm)


_sc_scatter = functools.partial(
    pl.kernel,
    out_type=jax.ShapeDtypeStruct((2, NSLOT, OPS), jnp.float32),
    mesh=plsc.VectorSubcoreMesh(core_axis_name="c", subcore_axis_name="s"),
    compiler_params=pltpu.CompilerParams(
        needs_layout_passes=False, use_tc_tiling_on_sc=False),
    scratch_types=[
        pltpu.VMEM((S // GB, GB), jnp.int32),
        pltpu.VMEM((CH,), jnp.int32),
        pltpu.VMEM((CH,), jnp.int32),
        pltpu.VMEM((GB, OPS), jnp.float32),
        pltpu.SemaphoreType.DMA,
    ],
)(_sc_body)


def kernel(alphas, noise_u, idx):
    idx = idx.astype(jnp.int32)
    opw_pad, alph_pad = _sample_and_pad(alphas, noise_u)
    out = _sc_scatter(idx[0], idx[1], opw_pad, alph_pad)
    return out.reshape(2, NN, NN, OPS)
